# Initial kernel scaffold; baseline (speedup 1.0000x reference)
import functools

import jax
import jax.numpy as jnp
from jax import lax
from jax.experimental import pallas as pl
from jax.experimental.pallas import tpu as pltpu

B, N, K, C = 8, 4096, 512, 64
NB = 512
NBLK = N // NB


def _tc_body(probs_ref, logits_ref, x1_ref, y1_ref, x2_ref, y2_ref,
             ha_ref, pc_ref, cx1_ref, cy1_ref, cx2_ref, cy2_ref):
    p = probs_ref[0]
    kio = lax.broadcasted_iota(jnp.int32, (NB, K), 1)
    pmax = jnp.max(p, axis=-1, keepdims=True)
    ha = jnp.min(jnp.where(p == pmax, kio, K), axis=-1)
    ha_ref[0, 0, :] = ha

    l = logits_ref[0]
    cio = lax.broadcasted_iota(jnp.int32, (NB, C), 1)
    lmax = jnp.max(l, axis=-1, keepdims=True)
    pc_ref[0, 0, :] = jnp.min(jnp.where(l == lmax, cio, C), axis=-1)

    onehot = ha[:, None] == kio
    x1 = x1_ref[0, 0, :]
    y1 = y1_ref[0, 0, :]
    x2 = x2_ref[0, 0, :]
    y2 = y2_ref[0, 0, :]
    px1 = jnp.min(jnp.where(onehot, x1[:, None], 1.0), axis=0)
    py1 = jnp.min(jnp.where(onehot, y1[:, None], 1.0), axis=0)
    px2 = jnp.max(jnp.where(onehot, x2[:, None], 0.0), axis=0)
    py2 = jnp.max(jnp.where(onehot, y2[:, None], 0.0), axis=0)

    nb = pl.program_id(1)

    @pl.when(nb == 0)
    def _():
        cx1_ref[0, 0, :] = px1
        cy1_ref[0, 0, :] = py1
        cx2_ref[0, 0, :] = px2
        cy2_ref[0, 0, :] = py2

    @pl.when(nb != 0)
    def _():
        cx1_ref[0, 0, :] = jnp.minimum(cx1_ref[0, 0, :], px1)
        cy1_ref[0, 0, :] = jnp.minimum(cy1_ref[0, 0, :], py1)
        cx2_ref[0, 0, :] = jnp.maximum(cx2_ref[0, 0, :], px2)
        cy2_ref[0, 0, :] = jnp.maximum(cy2_ref[0, 0, :], py2)


def _tc_call(probs, logits, x1, y1, x2, y2):
    tok3 = pl.BlockSpec((1, 1, NB), lambda b, i: (b, i, 0))
    comp3 = pl.BlockSpec((1, 1, K), lambda b, i: (b, 0, 0))
    return pl.pallas_call(
        _tc_body,
        grid=(B, NBLK),
        in_specs=[
            pl.BlockSpec((1, NB, K), lambda b, i: (b, i, 0)),
            pl.BlockSpec((1, NB, C), lambda b, i: (b, i, 0)),
            tok3, tok3, tok3, tok3,
        ],
        out_specs=[tok3, tok3, comp3, comp3, comp3, comp3],
        out_shape=[
            jax.ShapeDtypeStruct((B, NBLK, NB), jnp.int32),
            jax.ShapeDtypeStruct((B, NBLK, NB), jnp.int32),
            jax.ShapeDtypeStruct((B, 1, K), jnp.float32),
            jax.ShapeDtypeStruct((B, 1, K), jnp.float32),
            jax.ShapeDtypeStruct((B, 1, K), jnp.float32),
            jax.ShapeDtypeStruct((B, 1, K), jnp.float32),
        ],
        compiler_params=pltpu.CompilerParams(
            dimension_semantics=("parallel", "arbitrary"),
        ),
    )(probs, logits, x1, y1, x2, y2)


def kernel(boxes, assign_probs, class_logits):
    x1 = boxes[:, :, 0].reshape(B, NBLK, NB)
    y1 = boxes[:, :, 1].reshape(B, NBLK, NB)
    x2 = boxes[:, :, 2].reshape(B, NBLK, NB)
    y2 = boxes[:, :, 3].reshape(B, NBLK, NB)
    ha3, pc3, cx1, cy1, cx2, cy2 = _tc_call(
        assign_probs, class_logits, x1, y1, x2, y2)
    hard_assign = ha3.reshape(B, N)
    pred_classes = pc3.reshape(B, N)
    comp = jnp.stack(
        [cx1[:, 0, :], cy1[:, 0, :], cx2[:, 0, :], cy2[:, 0, :]], axis=-1)
    keep = jnp.ones((B, N), dtype=bool)
    return (hard_assign, pred_classes, boxes, keep, comp)


# speedup vs baseline: 2.3185x; 2.3185x over previous
import functools

import jax
import jax.numpy as jnp
from jax import lax
from jax.experimental import pallas as pl
from jax.experimental.pallas import tpu as pltpu

B, N, K, C = 8, 4096, 512, 64
NB = 512
NBLK = N // NB


def _tc_body(probs_ref, logits_ref, x1_ref, y1_ref, x2_ref, y2_ref,
             ha_ref, pc_ref, cx1_ref, cy1_ref, cx2_ref, cy2_ref):
    p = probs_ref[0]
    kio = lax.broadcasted_iota(jnp.int32, (NB, K), 1)
    pmax = jnp.max(p, axis=-1, keepdims=True)
    ha = jnp.min(jnp.where(p == pmax, kio, K), axis=-1)
    ha_ref[0, 0, 0, :] = ha

    l = logits_ref[0]
    cio = lax.broadcasted_iota(jnp.int32, (NB, C), 1)
    lmax = jnp.max(l, axis=-1, keepdims=True)
    pc_ref[0, 0, 0, :] = jnp.min(jnp.where(l == lmax, cio, C), axis=-1)

    onehot = ha[:, None] == kio
    x1 = x1_ref[0, 0, 0, :]
    y1 = y1_ref[0, 0, 0, :]
    x2 = x2_ref[0, 0, 0, :]
    y2 = y2_ref[0, 0, 0, :]
    px1 = jnp.min(jnp.where(onehot, x1[:, None], 1.0), axis=0)
    py1 = jnp.min(jnp.where(onehot, y1[:, None], 1.0), axis=0)
    px2 = jnp.max(jnp.where(onehot, x2[:, None], 0.0), axis=0)
    py2 = jnp.max(jnp.where(onehot, y2[:, None], 0.0), axis=0)

    nb = pl.program_id(1)

    @pl.when(nb == 0)
    def _():
        cx1_ref[0, 0, :] = px1
        cy1_ref[0, 0, :] = py1
        cx2_ref[0, 0, :] = px2
        cy2_ref[0, 0, :] = py2

    @pl.when(nb != 0)
    def _():
        cx1_ref[0, 0, :] = jnp.minimum(cx1_ref[0, 0, :], px1)
        cy1_ref[0, 0, :] = jnp.minimum(cy1_ref[0, 0, :], py1)
        cx2_ref[0, 0, :] = jnp.maximum(cx2_ref[0, 0, :], px2)
        cy2_ref[0, 0, :] = jnp.maximum(cy2_ref[0, 0, :], py2)


def _tc_call(probs, logits, x1, y1, x2, y2):
    tok3 = pl.BlockSpec((1, 1, 1, NB), lambda b, i: (b, i, 0, 0))
    comp3 = pl.BlockSpec((1, 1, K), lambda b, i: (b, 0, 0))
    return pl.pallas_call(
        _tc_body,
        grid=(B, NBLK),
        in_specs=[
            pl.BlockSpec((1, NB, K), lambda b, i: (b, i, 0)),
            pl.BlockSpec((1, NB, C), lambda b, i: (b, i, 0)),
            tok3, tok3, tok3, tok3,
        ],
        out_specs=[tok3, tok3, comp3, comp3, comp3, comp3],
        out_shape=[
            jax.ShapeDtypeStruct((B, NBLK, 1, NB), jnp.int32),
            jax.ShapeDtypeStruct((B, NBLK, 1, NB), jnp.int32),
            jax.ShapeDtypeStruct((B, 1, K), jnp.float32),
            jax.ShapeDtypeStruct((B, 1, K), jnp.float32),
            jax.ShapeDtypeStruct((B, 1, K), jnp.float32),
            jax.ShapeDtypeStruct((B, 1, K), jnp.float32),
        ],
        compiler_params=pltpu.CompilerParams(
            dimension_semantics=("parallel", "arbitrary"),
        ),
    )(probs, logits, x1, y1, x2, y2)


def kernel(boxes, assign_probs, class_logits):
    x1 = boxes[:, :, 0].reshape(B, NBLK, 1, NB)
    y1 = boxes[:, :, 1].reshape(B, NBLK, 1, NB)
    x2 = boxes[:, :, 2].reshape(B, NBLK, 1, NB)
    y2 = boxes[:, :, 3].reshape(B, NBLK, 1, NB)
    ha3, pc3, cx1, cy1, cx2, cy2 = _tc_call(
        assign_probs, class_logits, x1, y1, x2, y2)
    hard_assign = ha3.reshape(B, N)
    pred_classes = pc3.reshape(B, N)
    comp = jnp.stack(
        [cx1[:, 0, :], cy1[:, 0, :], cx2[:, 0, :], cy2[:, 0, :]], axis=-1)
    keep = jnp.ones((B, N), dtype=bool)
    return (hard_assign, pred_classes, boxes, keep, comp)
